# baseline (speedup 1.0000x reference)
import jax
import jax.numpy as jnp
from jax.experimental import pallas as pl
from jax.experimental.pallas import tpu as pltpu

VOCAB = 40
EMB = 32
S0, S1, S2 = 512, 1024, 2048
LATENT = 512
B, T = 256, 128
NT = 4
RT = B // NT
OHV = 128


def _gru_body(thr_ref, seq_ref, len_ref, perm_ref,
              bg0r, bc0r, bg1r, bc1r, bg2r, bc2r, bfr,
              g0a, c0a, g1a, c1a, g2a, c2a, w0a, w1a, w2a,
              out_ref,
              g0w, c0w, g1w, c1w, g2w, c2w, w0, w1, w2,
              xh0, xh1, xh2, h0, h1, h2, sems):
    f32 = jnp.float32
    bf16 = jnp.bfloat16

    srcs = (g0a, c0a, g1a, c1a, g2a, c2a, w0a, w1a, w2a)
    dsts = (g0w, c0w, g1w, c1w, g2w, c2w, w0, w1, w2)
    for i, (s, d) in enumerate(zip(srcs, dsts)):
        pltpu.make_async_copy(s, d, sems.at[i]).start()
    for i, (s, d) in enumerate(zip(srcs, dsts)):
        pltpu.make_async_copy(s, d, sems.at[i]).wait()

    h0[...] = jnp.zeros((B, S0), f32)
    h1[...] = jnp.zeros((B, S1), f32)
    h2[...] = jnp.zeros((B, S2), f32)
    xh0[:, OHV:] = jnp.zeros((B, S0), bf16)
    xh1[:, S0:] = jnp.zeros((B, S1), bf16)
    xh2[:, S1:] = jnp.zeros((B, S2), bf16)

    def cell(xh, ins, h_ref, gw, cw, bg, bc, outs, xb):
        xh[:, :ins] = xb
        h = h_ref[...]
        gpr = jnp.dot(xh[...], gw[:, :outs], preferred_element_type=f32) + bg[:, :outs]
        gpz = jnp.dot(xh[...], gw[:, outs:], preferred_element_type=f32) + bg[:, outs:]
        r = 0.5 * jnp.tanh(0.5 * gpr) + 0.5
        xh[:, ins:] = (r * h).astype(bf16)
        c = jnp.tanh(jnp.dot(xh[...], cw[...], preferred_element_type=f32) + bc[...])
        z = 0.5 * jnp.tanh(0.5 * gpz) + 0.5
        return c + z * (h - c)

    def make_phase(mp):
        viota = jax.lax.broadcasted_iota(jnp.int32, (OHV, mp), 0)

        def run(t, ids):
            ohT = jnp.where(ids[:, :mp] == viota, 1.0, 0.0)
            oh = jnp.transpose(ohT).astype(bf16)
            n0 = cell(xh0.at[:mp], OHV, h0.at[:mp], g0w, c0w, bg0r, bc0r, S0, oh)
            n1 = cell(xh1.at[:mp], S0, h1.at[:mp], g1w, c1w, bg1r, bc1r, S1,
                      n0.astype(bf16))
            n2 = cell(xh2.at[:mp], S1, h2.at[:mp], g2w, c2w, bg2r, bc2r, S2,
                      n1.astype(bf16))
            m = len_ref[:mp, :] > t
            hn0 = jnp.where(m, n0, h0[:mp, :])
            hn1 = jnp.where(m, n1, h1[:mp, :])
            hn2 = jnp.where(m, n2, h2[:mp, :])
            h0[:mp, :] = hn0
            h1[:mp, :] = hn1
            h2[:mp, :] = hn2
            xh0[:mp, OHV:] = hn0.astype(bf16)
            xh1[:mp, S0:] = hn1.astype(bf16)
            xh2[:mp, S1:] = hn2.astype(bf16)
        return run

    phases = [make_phase(RT * (q + 1)) for q in range(NT)]

    def step(t, _):
        ids = seq_ref[t]
        for q in range(NT):
            mp_active = t < thr_ref[q, 0]
            next_inactive = (t >= thr_ref[q + 1, 0]) if q + 1 < NT else True
            @pl.when(mp_active & next_inactive)
            def _(q=q):
                phases[q](t, ids)
        return 0

    jax.lax.fori_loop(0, T, step, 0)

    piota = jax.lax.broadcasted_iota(jnp.int32, (B, B), 0)
    P = jnp.where(perm_ref[...] == piota, 1.0, 0.0).astype(bf16)
    hq0 = jnp.dot(P, h0[...].astype(bf16), preferred_element_type=f32)
    hq1 = jnp.dot(P, h1[...].astype(bf16), preferred_element_type=f32)
    hq2 = jnp.dot(P, h2[...].astype(bf16), preferred_element_type=f32)
    p = (jnp.dot(hq0.astype(bf16), w0[...], preferred_element_type=f32)
         + jnp.dot(hq1.astype(bf16), w1[...], preferred_element_type=f32)
         + jnp.dot(hq2.astype(bf16), w2[...], preferred_element_type=f32)
         + bfr[...])
    out_ref[...] = jnp.tanh(p)


def kernel(input_seqs, input_lens, emb, Kg0, bg0, Kc0, bc0, Kg1, bg1, Kc1, bc1,
           Kg2, bg2, Kc2, bc2, W, b):
    f32 = jnp.float32
    bf16 = jnp.bfloat16

    def l0(K):
        top = jnp.zeros((OHV, K.shape[1]), f32).at[:VOCAB].set(emb.astype(f32) @ K[:EMB])
        return jnp.concatenate([top, K[EMB:]], axis=0).astype(bf16)

    g0w = l0(Kg0)
    c0w = l0(Kc0)
    g1w = Kg1.astype(bf16)
    c1w = Kc1.astype(bf16)
    g2w = Kg2.astype(bf16)
    c2w = Kc2.astype(bf16)
    wt = jnp.transpose(W)
    w0t = wt[:S0].astype(bf16)
    w1t = wt[S0:S0 + S1].astype(bf16)
    w2t = wt[S0 + S1:].astype(bf16)

    order = jnp.argsort(-input_lens).astype(jnp.int32)
    seqs_s = jnp.take(input_seqs, order, axis=0)
    lens_s = jnp.take(input_lens, order)
    thr = lens_s[:: RT].reshape(NT, 1)
    perm = order.reshape(1, B)

    seqs = jnp.transpose(seqs_s).reshape(T, 1, B)
    lens = lens_s.reshape(B, 1)
    bg0r = bg0.reshape(1, -1)
    bc0r = bc0.reshape(1, -1)
    bg1r = bg1.reshape(1, -1)
    bc1r = bc1.reshape(1, -1)
    bg2r = bg2.reshape(1, -1)
    bc2r = bc2.reshape(1, -1)
    bfr = b.reshape(1, -1)

    full = lambda shape: pl.BlockSpec(shape, lambda i: tuple(0 for _ in shape))
    anyspec = pl.BlockSpec(memory_space=pl.ANY)

    out = pl.pallas_call(
        _gru_body,
        grid=(1,),
        in_specs=[
            pl.BlockSpec(memory_space=pltpu.SMEM),
            pl.BlockSpec((T, 1, B), lambda i: (0, 0, 0)),
            pl.BlockSpec((B, 1), lambda i: (0, 0)),
            pl.BlockSpec((1, B), lambda i: (0, 0)),
            full((1, 2 * S0)), full((1, S0)),
            full((1, 2 * S1)), full((1, S1)),
            full((1, 2 * S2)), full((1, S2)),
            full((1, LATENT)),
        ] + [anyspec] * 9,
        out_specs=pl.BlockSpec((B, LATENT), lambda i: (0, 0)),
        out_shape=jax.ShapeDtypeStruct((B, LATENT), f32),
        scratch_shapes=[
            pltpu.VMEM((OHV + S0, 2 * S0), bf16),
            pltpu.VMEM((OHV + S0, S0), bf16),
            pltpu.VMEM((S0 + S1, 2 * S1), bf16),
            pltpu.VMEM((S0 + S1, S1), bf16),
            pltpu.VMEM((S1 + S2, 2 * S2), bf16),
            pltpu.VMEM((S1 + S2, S2), bf16),
            pltpu.VMEM((S0, LATENT), bf16),
            pltpu.VMEM((S1, LATENT), bf16),
            pltpu.VMEM((S2, LATENT), bf16),
            pltpu.VMEM((B, OHV + S0), bf16),
            pltpu.VMEM((B, S0 + S1), bf16),
            pltpu.VMEM((B, S1 + S2), bf16),
            pltpu.VMEM((B, S0), f32),
            pltpu.VMEM((B, S1), f32),
            pltpu.VMEM((B, S2), f32),
            pltpu.SemaphoreType.DMA((9,)),
        ],
        compiler_params=pltpu.CompilerParams(
            dimension_semantics=("arbitrary",),
            vmem_limit_bytes=64 * 1024 * 1024,
        ),
        name="mini_cddd_gru",
    )(thr, seqs, lens, perm, bg0r, bc0r, bg1r, bc1r, bg2r, bc2r, bfr,
      g0w, c0w, g1w, c1w, g2w, c2w, w0t, w1t, w2t)
    return out


# speedup vs baseline: 1.0083x; 1.0083x over previous
import jax
import jax.numpy as jnp
from jax.experimental import pallas as pl
from jax.experimental.pallas import tpu as pltpu

VOCAB = 40
EMB = 32
S0, S1, S2 = 512, 1024, 2048
LATENT = 512
B, T = 256, 128
NT = 4
RT = B // NT
OHV = 128


def _gru_body(thr_ref, seq_ref, len_ref, perm_ref,
              bg0r, bc0r, bg1r, bc1r, bg2r, bc2r, bfr,
              g0a, c0a, g1a, c1a, g2a, c2a, w0a, w1a, w2a,
              out_ref,
              g0w, c0w, g1w, c1w, g2w, c2w, w0, w1, w2,
              xh0, xh1, xh2, h0, h1, h2, sems):
    f32 = jnp.float32
    bf16 = jnp.bfloat16

    srcs = (g0a, c0a, g1a, c1a, g2a, c2a, w0a, w1a, w2a)
    dsts = (g0w, c0w, g1w, c1w, g2w, c2w, w0, w1, w2)
    for i, (s, d) in enumerate(zip(srcs, dsts)):
        pltpu.make_async_copy(s, d, sems.at[i]).start()
    for i, (s, d) in enumerate(zip(srcs, dsts)):
        pltpu.make_async_copy(s, d, sems.at[i]).wait()

    h0[...] = jnp.zeros((B, S0), f32)
    h1[...] = jnp.zeros((B, S1), f32)
    h2[...] = jnp.zeros((B, S2), f32)

    def cell(xh, ins, h_ref, gw, cw, bg, bc, outs, xb):
        xh[:, :ins] = xb
        h = h_ref[...]
        xh[:, ins:] = h.astype(bf16)
        gp = jnp.dot(xh[...], gw[...], preferred_element_type=f32) + bg[...]
        r = 0.5 * jnp.tanh(0.5 * gp[:, :outs]) + 0.5
        z = 0.5 * jnp.tanh(0.5 * gp[:, outs:]) + 0.5
        xh[:, ins:] = (r * h).astype(bf16)
        c = jnp.tanh(jnp.dot(xh[...], cw[...], preferred_element_type=f32) + bc[...])
        return c + z * (h - c)

    def make_phase(mp):
        viota = jax.lax.broadcasted_iota(jnp.int32, (OHV, mp), 0)

        def run(t, ids):
            ohT = jnp.where(ids[:, :mp] == viota, 1.0, 0.0)
            oh = jnp.transpose(ohT).astype(bf16)
            n0 = cell(xh0.at[:mp], OHV, h0.at[:mp], g0w, c0w, bg0r, bc0r, S0, oh)
            n1 = cell(xh1.at[:mp], S0, h1.at[:mp], g1w, c1w, bg1r, bc1r, S1,
                      n0.astype(bf16))
            n2 = cell(xh2.at[:mp], S1, h2.at[:mp], g2w, c2w, bg2r, bc2r, S2,
                      n1.astype(bf16))
            m = len_ref[:mp, :] > t
            h0[:mp, :] = jnp.where(m, n0, h0[:mp, :])
            h1[:mp, :] = jnp.where(m, n1, h1[:mp, :])
            h2[:mp, :] = jnp.where(m, n2, h2[:mp, :])
        return run

    phases = [make_phase(RT * (q + 1)) for q in range(NT)]

    def step(t, _):
        ids = seq_ref[t]
        for q in range(NT):
            mp_active = t < thr_ref[q, 0]
            next_inactive = (t >= thr_ref[q + 1, 0]) if q + 1 < NT else True
            @pl.when(mp_active & next_inactive)
            def _(q=q):
                phases[q](t, ids)
        return 0

    jax.lax.fori_loop(0, T, step, 0)

    piota = jax.lax.broadcasted_iota(jnp.int32, (B, B), 0)
    P = jnp.where(perm_ref[...] == piota, 1.0, 0.0).astype(bf16)
    hq0 = jnp.dot(P, h0[...].astype(bf16), preferred_element_type=f32)
    hq1 = jnp.dot(P, h1[...].astype(bf16), preferred_element_type=f32)
    hq2 = jnp.dot(P, h2[...].astype(bf16), preferred_element_type=f32)
    p = (jnp.dot(hq0.astype(bf16), w0[...], preferred_element_type=f32)
         + jnp.dot(hq1.astype(bf16), w1[...], preferred_element_type=f32)
         + jnp.dot(hq2.astype(bf16), w2[...], preferred_element_type=f32)
         + bfr[...])
    out_ref[...] = jnp.tanh(p)


def kernel(input_seqs, input_lens, emb, Kg0, bg0, Kc0, bc0, Kg1, bg1, Kc1, bc1,
           Kg2, bg2, Kc2, bc2, W, b):
    f32 = jnp.float32
    bf16 = jnp.bfloat16

    def l0(K):
        top = jnp.zeros((OHV, K.shape[1]), f32).at[:VOCAB].set(emb.astype(f32) @ K[:EMB])
        return jnp.concatenate([top, K[EMB:]], axis=0).astype(bf16)

    g0w = l0(Kg0)
    c0w = l0(Kc0)
    g1w = Kg1.astype(bf16)
    c1w = Kc1.astype(bf16)
    g2w = Kg2.astype(bf16)
    c2w = Kc2.astype(bf16)
    wt = jnp.transpose(W)
    w0t = wt[:S0].astype(bf16)
    w1t = wt[S0:S0 + S1].astype(bf16)
    w2t = wt[S0 + S1:].astype(bf16)

    order = jnp.argsort(-input_lens).astype(jnp.int32)
    seqs_s = jnp.take(input_seqs, order, axis=0)
    lens_s = jnp.take(input_lens, order)
    thr = lens_s[:: RT].reshape(NT, 1)
    perm = order.reshape(1, B)

    seqs = jnp.transpose(seqs_s).reshape(T, 1, B)
    lens = lens_s.reshape(B, 1)
    bg0r = bg0.reshape(1, -1)
    bc0r = bc0.reshape(1, -1)
    bg1r = bg1.reshape(1, -1)
    bc1r = bc1.reshape(1, -1)
    bg2r = bg2.reshape(1, -1)
    bc2r = bc2.reshape(1, -1)
    bfr = b.reshape(1, -1)

    full = lambda shape: pl.BlockSpec(shape, lambda i: tuple(0 for _ in shape))
    anyspec = pl.BlockSpec(memory_space=pl.ANY)

    out = pl.pallas_call(
        _gru_body,
        grid=(1,),
        in_specs=[
            pl.BlockSpec(memory_space=pltpu.SMEM),
            pl.BlockSpec((T, 1, B), lambda i: (0, 0, 0)),
            pl.BlockSpec((B, 1), lambda i: (0, 0)),
            pl.BlockSpec((1, B), lambda i: (0, 0)),
            full((1, 2 * S0)), full((1, S0)),
            full((1, 2 * S1)), full((1, S1)),
            full((1, 2 * S2)), full((1, S2)),
            full((1, LATENT)),
        ] + [anyspec] * 9,
        out_specs=pl.BlockSpec((B, LATENT), lambda i: (0, 0)),
        out_shape=jax.ShapeDtypeStruct((B, LATENT), f32),
        scratch_shapes=[
            pltpu.VMEM((OHV + S0, 2 * S0), bf16),
            pltpu.VMEM((OHV + S0, S0), bf16),
            pltpu.VMEM((S0 + S1, 2 * S1), bf16),
            pltpu.VMEM((S0 + S1, S1), bf16),
            pltpu.VMEM((S1 + S2, 2 * S2), bf16),
            pltpu.VMEM((S1 + S2, S2), bf16),
            pltpu.VMEM((S0, LATENT), bf16),
            pltpu.VMEM((S1, LATENT), bf16),
            pltpu.VMEM((S2, LATENT), bf16),
            pltpu.VMEM((B, OHV + S0), bf16),
            pltpu.VMEM((B, S0 + S1), bf16),
            pltpu.VMEM((B, S1 + S2), bf16),
            pltpu.VMEM((B, S0), f32),
            pltpu.VMEM((B, S1), f32),
            pltpu.VMEM((B, S2), f32),
            pltpu.SemaphoreType.DMA((9,)),
        ],
        compiler_params=pltpu.CompilerParams(
            dimension_semantics=("arbitrary",),
            vmem_limit_bytes=64 * 1024 * 1024,
        ),
        name="mini_cddd_gru",
    )(thr, seqs, lens, perm, bg0r, bc0r, bg1r, bc1r, bg2r, bc2r, bfr,
      g0w, c0w, g1w, c1w, g2w, c2w, w0t, w1t, w2t)
    return out
